# Initial kernel scaffold; baseline (speedup 1.0000x reference)
import jax
import jax.numpy as jnp
from jax.experimental import pallas as pl


def kernel(scores, k):
    raise NotImplementedError("write your pallas kernel here")


# speedup vs baseline: 3.2034x; 3.2034x over previous
import jax
import jax.numpy as jnp
from jax.experimental import pallas as pl

_K = 64
_NEG = -1e30


def _topk_mask_body(x_ref, o_ref):
    x = x_ref[...]
    r, c = x.shape
    b = jax.lax.bitcast_convert_type(x, jnp.int32)
    key = b ^ ((b >> 31) & jnp.int32(0x7FFFFFFF))

    def count_ge(t):
        return jnp.sum((key >= t).astype(jnp.int32), axis=1, keepdims=True)

    lo = jnp.full((r, 1), -(2**31), jnp.int32)
    hi = jnp.full((r, 1), 2**31 - 1, jnp.int32)

    def body(_, lohi):
        lo, hi = lohi
        lop1 = lo + 1
        mid = (lop1 & hi) + ((lop1 ^ hi) >> 1)
        ge = count_ge(mid) >= _K
        return jnp.where(ge, mid, lo), jnp.where(ge, hi, mid - 1)

    lo, hi = jax.lax.fori_loop(0, 32, body, (lo, hi))
    t = lo

    m = _K - jnp.sum((key > t).astype(jnp.int32), axis=1, keepdims=True)
    eq = key == t
    col = jax.lax.broadcasted_iota(jnp.int32, (r, c), 1)
    lo2 = jnp.zeros((r, 1), jnp.int32)
    hi2 = jnp.full((r, 1), c - 1, jnp.int32)

    def body2(_, lohi):
        lo2, hi2 = lohi
        mid = (lo2 + hi2) >> 1
        cnt = jnp.sum((eq & (col <= mid)).astype(jnp.int32), axis=1, keepdims=True)
        ok = cnt >= m
        return jnp.where(ok, lo2, mid + 1), jnp.where(ok, mid, hi2)

    ti, _ = jax.lax.fori_loop(0, 15, body2, (lo2, hi2))

    keep = (key > t) | (eq & (col <= ti))
    o_ref[...] = jnp.where(keep, x, jnp.float32(_NEG))


def kernel(scores, k):
    r, c = scores.shape
    br = 8
    out = pl.pallas_call(
        _topk_mask_body,
        grid=(r // br,),
        in_specs=[pl.BlockSpec((br, c), lambda i: (i, 0))],
        out_specs=pl.BlockSpec((br, c), lambda i: (i, 0)),
        out_shape=jax.ShapeDtypeStruct((r, c), jnp.float32),
    )(scores)
    return out + (k * 0)


# speedup vs baseline: 3.6313x; 1.1336x over previous
import functools
import jax
import jax.numpy as jnp
from jax import lax
from jax.experimental import pallas as pl
from jax.experimental.pallas import tpu as pltpu
from jax.experimental.pallas import tpu_sc as plsc

_K = 64
_NEG = -1e30
_R, _C = 128, 32768
_NV = _C // 16
_GROUPS = 64
_VPG = _NV // _GROUPS
_ROWS_PER_TEC = 4


def _key(v):
    b = plsc.bitcast(v, jnp.int32)
    return b ^ ((b >> 31) & jnp.int32(0x7FFFFFFF))


def _unkey(k):
    return plsc.bitcast(k ^ ((k >> 31) & jnp.int32(0x7FFFFFFF)), jnp.float32)


def _last(v):
    return lax.reduce_max(v, axes=(0,))


def _sc_body(scores_hbm, out_hbm, rowbuf, candk, candi):
    wid = lax.axis_index("s") * 2 + lax.axis_index("c")
    lane = lax.iota(jnp.int32, 16)

    for r in range(_ROWS_PER_TEC):
        row = wid * _ROWS_PER_TEC + r
        pltpu.sync_copy(scores_hbm.at[row], rowbuf)

        def grp_body(g, t_low):
            def vb(i, acc):
                return jnp.maximum(acc, rowbuf[pl.ds(g * 512 + i * 16, 16)])
            gm = lax.fori_loop(0, _VPG, vb,
                               jnp.full((16,), -jnp.inf, jnp.float32))
            return jnp.minimum(t_low, lax.reduce_max(gm, axes=(0,)))
        t_low = lax.fori_loop(0, _GROUPS, grp_body, jnp.float32(jnp.inf))
        tlk = _key(jnp.full((16,), t_low, jnp.float32))

        def collect(i, cnt):
            kv = _key(rowbuf[pl.ds(i * 16, 16)])
            m = kv >= tlk
            inc = plsc.cumsum(m.astype(jnp.int32))
            pos = cnt + inc - 1
            plsc.store_scatter(candk, [pos], kv, mask=m)
            plsc.store_scatter(candi, [pos], lane + i * 16, mask=m)
            return cnt + _last(inc)
        cnt = lax.fori_loop(0, _NV, collect, jnp.int32(0))

        plsc.store_scatter(candk, [cnt + lane],
                           _key(jnp.full((16,), -jnp.inf, jnp.float32)))
        nv = (cnt + 15) // 16

        def count_ge(t):
            def cb(j, acc):
                return acc + (candk[pl.ds(j * 16, 16)] >= t).astype(jnp.int32)
            part = lax.fori_loop(0, nv, cb, jnp.zeros((16,), jnp.int32))
            return lax.reduce_max(plsc.cumsum(part), axes=(0,))

        def bs_body(_, lohi):
            lo, hi = lohi
            lop1 = lo + 1
            mid = (lop1 & hi) + ((lop1 ^ hi) >> 1)
            ge = count_ge(mid) >= _K
            return (jnp.where(ge, mid, lo), jnp.where(ge, hi, mid - 1))
        t, _unused = lax.fori_loop(
            0, 32, bs_body,
            (jnp.int32(-(2**31)), jnp.int32(2**31 - 1)))

        c_gt = count_ge(t + 1)
        m_eq = _K - c_gt

        def fill(i, _):
            rowbuf[pl.ds(i * 16, 16)] = jnp.full((16,), _NEG, jnp.float32)
            return 0
        lax.fori_loop(0, _NV, fill, 0)

        def emit(j, eq_seen):
            kv = candk[pl.ds(j * 16, 16)]
            iv = candi[pl.ds(j * 16, 16)]
            gt = kv > t
            eq = kv == t
            inc = plsc.cumsum(eq.astype(jnp.int32))
            keep = gt | (eq & ((eq_seen + inc) <= m_eq))
            plsc.store_scatter(rowbuf, [iv], _unkey(kv), mask=keep)
            return eq_seen + _last(inc)
        lax.fori_loop(0, nv, emit, jnp.int32(0))

        pltpu.sync_copy(rowbuf, out_hbm.at[row])


def kernel(scores, k):
    mesh = plsc.VectorSubcoreMesh(core_axis_name="c", subcore_axis_name="s",
                                  num_cores=2, num_subcores=16)
    out = pl.kernel(
        _sc_body,
        out_type=jax.ShapeDtypeStruct((_R, _C), jnp.float32),
        mesh=mesh,
        compiler_params=pltpu.CompilerParams(needs_layout_passes=False),
        scratch_types=[
            pltpu.VMEM((_C,), jnp.float32),
            pltpu.VMEM((_C + 16,), jnp.int32),
            pltpu.VMEM((_C + 16,), jnp.int32),
        ],
    )(scores)
    return out + (k * 0)


# speedup vs baseline: 5.0426x; 1.3887x over previous
import jax
import jax.numpy as jnp
from jax import lax
from jax.experimental import pallas as pl
from jax.experimental.pallas import tpu as pltpu
from jax.experimental.pallas import tpu_sc as plsc

_K = 64
_NEG = -1e30
_R, _C = 128, 32768
_NV = _C // 16
_GROUPS = 64
_VPG = _NV // _GROUPS
_ROWS_PER_TEC = 4


def _key(v):
    b = plsc.bitcast(v, jnp.int32)
    return b ^ ((b >> 31) & jnp.int32(0x7FFFFFFF))


def _sc_body(scores_hbm, out_hbm, rowbuf, candk):
    wid = lax.axis_index("s") * 2 + lax.axis_index("c")
    neg = jnp.full((16,), _NEG, jnp.float32)

    for r in range(_ROWS_PER_TEC):
        row = wid * _ROWS_PER_TEC + r
        pltpu.sync_copy(scores_hbm.at[row], rowbuf)

        def grp_body(g, t_low):
            def vb(i, acc):
                accs = list(acc)
                for u in range(8):
                    accs[u] = jnp.maximum(
                        accs[u], rowbuf[pl.ds(g * 512 + (i * 8 + u) * 16, 16)])
                return tuple(accs)
            gms = lax.fori_loop(
                0, _VPG // 8, vb,
                tuple(jnp.full((16,), -jnp.inf, jnp.float32) for _ in range(8)))
            gm = gms[0]
            for u in range(1, 8):
                gm = jnp.maximum(gm, gms[u])
            return jnp.minimum(t_low, lax.reduce_max(gm, axes=(0,)))
        t_low = lax.fori_loop(0, _GROUPS, grp_body, jnp.float32(jnp.inf))
        tlk = _key(jnp.full((16,), t_low, jnp.float32))

        def collect(i, cnt):
            for u in range(4):
                kv = _key(rowbuf[pl.ds((i * 4 + u) * 16, 16)])
                m = kv >= tlk
                plsc.store_compressed(candk.at[pl.ds(cnt, 16)], kv, mask=m)
                cnt = cnt + plsc.all_reduce_population_count(m)[0]
            return cnt
        cnt = lax.fori_loop(0, _NV // 4, collect, jnp.int32(0))

        plsc.store_compressed(
            candk.at[pl.ds(cnt, 16)],
            _key(jnp.full((16,), -jnp.inf, jnp.float32)),
            mask=jnp.full((16,), True, jnp.bool_))
        nv = (cnt + 15) // 16

        def count_ge(t):
            def cb(j, acc):
                return acc + (candk[pl.ds(j * 16, 16)] >= t).astype(jnp.int32)
            part = lax.fori_loop(0, nv, cb, jnp.zeros((16,), jnp.int32))
            return lax.reduce_max(plsc.cumsum(part), axes=(0,))

        def bs_body(_, lohi):
            lo, hi = lohi
            lop1 = lo + 1
            mid = (lop1 & hi) + ((lop1 ^ hi) >> 1)
            ge = count_ge(mid) >= _K
            return (jnp.where(ge, mid, lo), jnp.where(ge, hi, mid - 1))
        t, _unused = lax.fori_loop(
            0, 32, bs_body,
            (jnp.int32(-(2**31)), jnp.int32(2**31 - 1)))

        m_eq = _K - count_ge(t + 1)

        def emit(i, eq_seen):
            for u in range(4):
                sl = pl.ds((i * 4 + u) * 16, 16)
                x = rowbuf[sl]
                kv = _key(x)
                eq = kv == t
                inc = plsc.cumsum(eq.astype(jnp.int32))
                keep = (kv > t) | (eq & ((eq_seen + inc) <= m_eq))
                rowbuf[sl] = jnp.where(keep, x, neg)
                eq_seen = eq_seen + plsc.all_reduce_population_count(eq)[0]
            return eq_seen
        lax.fori_loop(0, _NV // 4, emit, jnp.int32(0))

        pltpu.sync_copy(rowbuf, out_hbm.at[row])


def kernel(scores, k):
    mesh = plsc.VectorSubcoreMesh(core_axis_name="c", subcore_axis_name="s",
                                  num_cores=2, num_subcores=16)
    out = pl.kernel(
        _sc_body,
        out_type=jax.ShapeDtypeStruct((_R, _C), jnp.float32),
        mesh=mesh,
        compiler_params=pltpu.CompilerParams(needs_layout_passes=False),
        scratch_types=[
            pltpu.VMEM((_C,), jnp.float32),
            pltpu.VMEM((_C + 16,), jnp.int32),
        ],
    )(scores)
    return out + (k * 0)


# speedup vs baseline: 8.5941x; 1.7043x over previous
import jax
import jax.numpy as jnp
from jax import lax
from jax.experimental import pallas as pl
from jax.experimental.pallas import tpu as pltpu
from jax.experimental.pallas import tpu_sc as plsc

_K = 64
_NEG = -1e30
_R, _C = 128, 32768
_NV = _C // 16
_GROUPS = 64
_VPG = _NV // _GROUPS
_ROWS_PER_TEC = 4


def _key(v):
    b = plsc.bitcast(v, jnp.int32)
    return b ^ ((b >> 31) & jnp.int32(0x7FFFFFFF))


def _sc_body(scores_hbm, out_hbm, rowa, rowb, candk,
             in_sem_a, in_sem_b, out_sem_a, out_sem_b):
    wid = lax.axis_index("s") * 2 + lax.axis_index("c")
    neg = jnp.full((16,), _NEG, jnp.float32)
    bufs = [rowa, rowb]
    in_sems = [in_sem_a, in_sem_b]
    out_sems = [out_sem_a, out_sem_b]

    base_row = wid * _ROWS_PER_TEC
    in_flight = {0: pltpu.async_copy(scores_hbm.at[base_row], rowa, in_sem_a)}
    out_flight = {}

    for r in range(_ROWS_PER_TEC):
        rowbuf = bufs[r % 2]
        in_flight.pop(r).wait()

        def grp_body(g, t_low):
            def vb(i, acc):
                accs = list(acc)
                for u in range(8):
                    accs[u] = jnp.maximum(
                        accs[u], rowbuf[pl.ds(g * 512 + (i * 8 + u) * 16, 16)])
                return tuple(accs)
            gms = lax.fori_loop(
                0, _VPG // 8, vb,
                tuple(jnp.full((16,), -jnp.inf, jnp.float32) for _ in range(8)))
            gm = gms[0]
            for u in range(1, 8):
                gm = jnp.maximum(gm, gms[u])
            return jnp.minimum(t_low, lax.reduce_max(gm, axes=(0,)))
        t_low = lax.fori_loop(0, _GROUPS, grp_body, jnp.float32(jnp.inf))
        tlk = _key(jnp.full((16,), t_low, jnp.float32))

        if r + 1 < _ROWS_PER_TEC:
            nxt = (r + 1) % 2
            if r - 1 in out_flight:
                out_flight.pop(r - 1).wait()
            in_flight[r + 1] = pltpu.async_copy(
                scores_hbm.at[base_row + r + 1], bufs[nxt], in_sems[nxt])

        def collect(i, cnt):
            kvs, msks, pcs = [], [], []
            for u in range(4):
                kv = _key(rowbuf[pl.ds((i * 4 + u) * 16, 16)])
                m = kv >= tlk
                kvs.append(kv)
                msks.append(m)
                pcs.append(plsc.all_reduce_population_count(m)[0])
            offs = [cnt]
            for u in range(3):
                offs.append(offs[u] + pcs[u])
            for u in range(4):
                plsc.store_compressed(candk.at[pl.ds(offs[u], 16)],
                                      kvs[u], mask=msks[u])
            return offs[3] + pcs[3]
        cnt = lax.fori_loop(0, _NV // 4, collect, jnp.int32(0))

        plsc.store_compressed(
            candk.at[pl.ds(cnt, 16)],
            _key(jnp.full((16,), -jnp.inf, jnp.float32)),
            mask=jnp.full((16,), True, jnp.bool_))
        nv = (cnt + 15) // 16

        def count_ge(t):
            def cb(j, acc):
                return acc + (candk[pl.ds(j * 16, 16)] >= t).astype(jnp.int32)
            part = lax.fori_loop(0, nv, cb, jnp.zeros((16,), jnp.int32))
            return lax.reduce_max(plsc.cumsum(part), axes=(0,))

        def bs_body(_, lohi):
            lo, hi = lohi
            lop1 = lo + 1
            mid = (lop1 & hi) + ((lop1 ^ hi) >> 1)
            ge = count_ge(mid) >= _K
            return (jnp.where(ge, mid, lo), jnp.where(ge, hi, mid - 1))
        t, _unused = lax.fori_loop(
            0, 32, bs_body,
            (jnp.int32(-(2**31)), jnp.int32(2**31 - 1)))

        total = count_ge(t)

        @pl.when(total == _K)
        def _():
            def emit_fast(i, carry):
                for u in range(4):
                    sl = pl.ds((i * 4 + u) * 16, 16)
                    x = rowbuf[sl]
                    rowbuf[sl] = jnp.where(_key(x) >= t, x, neg)
                return carry
            lax.fori_loop(0, _NV // 4, emit_fast, 0)

        @pl.when(total != _K)
        def _():
            m_eq = _K - count_ge(t + 1)

            def emit(i, eq_seen):
                for u in range(4):
                    sl = pl.ds((i * 4 + u) * 16, 16)
                    x = rowbuf[sl]
                    kv = _key(x)
                    eq = kv == t
                    inc = plsc.cumsum(eq.astype(jnp.int32))
                    keep = (kv > t) | (eq & ((eq_seen + inc) <= m_eq))
                    rowbuf[sl] = jnp.where(keep, x, neg)
                    eq_seen = eq_seen + plsc.all_reduce_population_count(eq)[0]
                return eq_seen
            lax.fori_loop(0, _NV // 4, emit, jnp.int32(0))

        out_flight[r] = pltpu.async_copy(
            rowbuf, out_hbm.at[base_row + r], out_sems[r % 2])

    out_flight.pop(_ROWS_PER_TEC - 2).wait()
    out_flight.pop(_ROWS_PER_TEC - 1).wait()


def kernel(scores, k):
    mesh = plsc.VectorSubcoreMesh(core_axis_name="c", subcore_axis_name="s",
                                  num_cores=2, num_subcores=16)
    out = pl.kernel(
        _sc_body,
        out_type=jax.ShapeDtypeStruct((_R, _C), jnp.float32),
        mesh=mesh,
        compiler_params=pltpu.CompilerParams(needs_layout_passes=False),
        scratch_types=[
            pltpu.VMEM((_C,), jnp.float32),
            pltpu.VMEM((_C,), jnp.float32),
            pltpu.VMEM((_C + 16,), jnp.int32),
            pltpu.SemaphoreType.DMA,
            pltpu.SemaphoreType.DMA,
            pltpu.SemaphoreType.DMA,
            pltpu.SemaphoreType.DMA,
        ],
    )(scores)
    return out + (k * 0)


# speedup vs baseline: 13.4615x; 1.5664x over previous
import jax
import jax.numpy as jnp
from jax import lax
from jax.experimental import pallas as pl
from jax.experimental.pallas import tpu as pltpu
from jax.experimental.pallas import tpu_sc as plsc

_K = 64
_NEG = -1e30
_R, _C = 128, 32768
_NV = _C // 16
_GROUPS = 64
_VPG = _NV // _GROUPS
_ROWS_PER_TEC = 4


def _key(v):
    b = plsc.bitcast(v, jnp.int32)
    return b ^ ((b >> 31) & jnp.int32(0x7FFFFFFF))


def _unkey_splat(t):
    ts = jnp.broadcast_to(t, (16,))
    return plsc.bitcast(ts ^ ((ts >> 31) & jnp.int32(0x7FFFFFFF)), jnp.float32)


def _scalar(v16):
    return lax.reduce_max(v16, axes=(0,))


def _sc_body(scores_hbm, out_hbm, rowa, rowb, candv,
             in_sem_a, in_sem_b, out_sem_a, out_sem_b):
    wid = lax.axis_index("s") * 2 + lax.axis_index("c")
    neg = jnp.full((16,), _NEG, jnp.float32)
    bufs = [rowa, rowb]
    in_sems = [in_sem_a, in_sem_b]
    out_sems = [out_sem_a, out_sem_b]

    base_row = wid * _ROWS_PER_TEC
    in_flight = {0: pltpu.async_copy(scores_hbm.at[base_row], rowa, in_sem_a)}
    out_flight = {}

    for r in range(_ROWS_PER_TEC):
        rowbuf = bufs[r % 2]
        in_flight.pop(r).wait()

        def grp_body(g, carry):
            t_low, gmax = carry

            def vb(i, acc):
                accs = list(acc)
                for u in range(8):
                    accs[u] = jnp.maximum(
                        accs[u], rowbuf[pl.ds(g * 512 + (i * 8 + u) * 16, 16)])
                return tuple(accs)
            gms = lax.fori_loop(
                0, _VPG // 8, vb,
                tuple(jnp.full((16,), -jnp.inf, jnp.float32) for _ in range(8)))
            gm = gms[0]
            for u in range(1, 8):
                gm = jnp.maximum(gm, gms[u])
            gmx = lax.reduce_max(gm, axes=(0,))
            return jnp.minimum(t_low, gmx), jnp.maximum(gmax, gmx)
        t_low, gmax = lax.fori_loop(
            0, _GROUPS, grp_body,
            (jnp.float32(jnp.inf), jnp.float32(-jnp.inf)))
        tl_vec = jnp.broadcast_to(t_low, (16,))

        if r + 1 < _ROWS_PER_TEC:
            nxt = (r + 1) % 2
            if r - 1 in out_flight:
                out_flight.pop(r - 1).wait()
            in_flight[r + 1] = pltpu.async_copy(
                scores_hbm.at[base_row + r + 1], bufs[nxt], in_sems[nxt])

        def collect(i, cnt):
            xs, msks, pcs = [], [], []
            for u in range(8):
                x = rowbuf[pl.ds((i * 8 + u) * 16, 16)]
                m = x >= tl_vec
                xs.append(x)
                msks.append(m)
                pcs.append(plsc.all_reduce_population_count(m)[0])
            offs = [cnt]
            for u in range(7):
                offs.append(offs[u] + pcs[u])
            for u in range(8):
                plsc.store_compressed(candv.at[pl.ds(offs[u], 16)],
                                      xs[u], mask=msks[u])
            return offs[7] + pcs[7]
        cnt = lax.fori_loop(0, _NV // 8, collect, jnp.int32(0))

        plsc.store_compressed(
            candv.at[pl.ds(cnt, 16)],
            jnp.full((16,), -jnp.inf, jnp.float32),
            mask=jnp.full((16,), True, jnp.bool_))
        nv = (cnt + 15) // 16

        def count_ge(t):
            def cb(j, acc):
                kv = _key(candv[pl.ds(j * 16, 16)])
                return acc + (kv >= t).astype(jnp.int32)
            part = lax.fori_loop(0, nv, cb, jnp.zeros((16,), jnp.int32))
            return lax.reduce_max(plsc.cumsum(part), axes=(0,))

        lo0 = _scalar(_key(tl_vec)) - 1
        hi0 = _scalar(_key(jnp.broadcast_to(gmax, (16,)))) + 1

        def bs_cond(lohi):
            return lohi[0] < lohi[1]

        def bs_body(lohi):
            lo, hi = lohi
            lop1 = lo + 1
            mid = (lop1 & hi) + ((lop1 ^ hi) >> 1)
            ge = count_ge(mid) >= _K
            return (jnp.where(ge, mid, lo), jnp.where(ge, hi, mid - 1))
        t, _unused = lax.while_loop(bs_cond, bs_body, (lo0, hi0))

        total = count_ge(t)
        fast = (total == _K) & (t != 0) & (t != -1)

        @pl.when(fast)
        def _():
            tf = _unkey_splat(t)

            def emit_fast(i, carry):
                for u in range(8):
                    sl = pl.ds((i * 8 + u) * 16, 16)
                    x = rowbuf[sl]
                    rowbuf[sl] = jnp.where(x >= tf, x, neg)
                return carry
            lax.fori_loop(0, _NV // 8, emit_fast, 0)

        @pl.when(jnp.logical_not(fast))
        def _():
            m_eq = _K - count_ge(t + 1)

            def emit(i, eq_seen):
                for u in range(4):
                    sl = pl.ds((i * 4 + u) * 16, 16)
                    x = rowbuf[sl]
                    kv = _key(x)
                    eq = kv == t
                    inc = plsc.cumsum(eq.astype(jnp.int32))
                    keep = (kv > t) | (eq & ((eq_seen + inc) <= m_eq))
                    rowbuf[sl] = jnp.where(keep, x, neg)
                    eq_seen = eq_seen + plsc.all_reduce_population_count(eq)[0]
                return eq_seen
            lax.fori_loop(0, _NV // 4, emit, jnp.int32(0))

        out_flight[r] = pltpu.async_copy(
            rowbuf, out_hbm.at[base_row + r], out_sems[r % 2])

    out_flight.pop(_ROWS_PER_TEC - 2).wait()
    out_flight.pop(_ROWS_PER_TEC - 1).wait()


def kernel(scores, k):
    mesh = plsc.VectorSubcoreMesh(core_axis_name="c", subcore_axis_name="s",
                                  num_cores=2, num_subcores=16)
    out = pl.kernel(
        _sc_body,
        out_type=jax.ShapeDtypeStruct((_R, _C), jnp.float32),
        mesh=mesh,
        compiler_params=pltpu.CompilerParams(needs_layout_passes=False),
        scratch_types=[
            pltpu.VMEM((_C,), jnp.float32),
            pltpu.VMEM((_C,), jnp.float32),
            pltpu.VMEM((_C + 16,), jnp.float32),
            pltpu.SemaphoreType.DMA,
            pltpu.SemaphoreType.DMA,
            pltpu.SemaphoreType.DMA,
            pltpu.SemaphoreType.DMA,
        ],
    )(scores)
    return out + (k * 0)


# speedup vs baseline: 16.5489x; 1.2293x over previous
import jax
import jax.numpy as jnp
from jax import lax
from jax.experimental import pallas as pl
from jax.experimental.pallas import tpu as pltpu
from jax.experimental.pallas import tpu_sc as plsc

_K = 64
_NEG = -1e30
_R, _C = 128, 32768
_NV = _C // 16
_GROUPS = 64
_VPG = _NV // _GROUPS
_ROWS_PER_TEC = 4


def _key(v):
    b = plsc.bitcast(v, jnp.int32)
    return b ^ ((b >> 31) & jnp.int32(0x7FFFFFFF))


def _unkey_splat(t):
    ts = jnp.broadcast_to(t, (16,))
    return plsc.bitcast(ts ^ ((ts >> 31) & jnp.int32(0x7FFFFFFF)), jnp.float32)


def _scalar(v16):
    return lax.reduce_max(v16, axes=(0,))


def _sc_body(scores_hbm, out_hbm, rowa, rowb, candv,
             in_sem_a, in_sem_b, out_sem_a, out_sem_b):
    wid = lax.axis_index("s") * 2 + lax.axis_index("c")
    neg = jnp.full((16,), _NEG, jnp.float32)
    bufs = [rowa, rowb]
    in_sems = [in_sem_a, in_sem_b]
    out_sems = [out_sem_a, out_sem_b]

    base_row = wid * _ROWS_PER_TEC
    in_flight = {0: pltpu.async_copy(scores_hbm.at[base_row], rowa, in_sem_a)}
    out_flight = {}

    for r in range(_ROWS_PER_TEC):
        rowbuf = bufs[r % 2]
        in_flight.pop(r).wait()

        def grp_body(g, carry):
            t_low, gmax = carry
            gms = [rowbuf[pl.ds(g * 512 + u * 16, 16)] for u in range(8)]
            for u in range(8, _VPG):
                gms[u % 8] = jnp.maximum(
                    gms[u % 8], rowbuf[pl.ds(g * 512 + u * 16, 16)])
            gm = gms[0]
            for u in range(1, 8):
                gm = jnp.maximum(gm, gms[u])
            gmx = lax.reduce_max(gm, axes=(0,))
            return jnp.minimum(t_low, gmx), jnp.maximum(gmax, gmx)
        t_low, gmax = lax.fori_loop(
            0, _GROUPS, grp_body,
            (jnp.float32(jnp.inf), jnp.float32(-jnp.inf)))
        tl_vec = jnp.broadcast_to(t_low, (16,))

        if r + 1 < _ROWS_PER_TEC:
            nxt = (r + 1) % 2
            if r - 1 in out_flight:
                out_flight.pop(r - 1).wait()
            in_flight[r + 1] = pltpu.async_copy(
                scores_hbm.at[base_row + r + 1], bufs[nxt], in_sems[nxt])

        def collect(i, cnt):
            xs, msks, pcs = [], [], []
            for u in range(16):
                x = rowbuf[pl.ds((i * 16 + u) * 16, 16)]
                m = x >= tl_vec
                xs.append(x)
                msks.append(m)
                pcs.append(plsc.all_reduce_population_count(m)[0])
            offs = [cnt]
            for u in range(15):
                offs.append(offs[u] + pcs[u])
            for u in range(16):
                plsc.store_compressed(candv.at[pl.ds(offs[u], 16)],
                                      xs[u], mask=msks[u])
            return offs[15] + pcs[15]
        cnt = lax.fori_loop(0, _NV // 16, collect, jnp.int32(0))

        inf_pad = jnp.full((16,), -jnp.inf, jnp.float32)
        true16 = jnp.full((16,), True, jnp.bool_)
        for p in range(4):
            plsc.store_compressed(candv.at[pl.ds(cnt + p * 16, 16)],
                                  inf_pad, mask=true16)
        nv4 = (cnt + 63) // 64

        def count_ge2(t1, t2):
            def cb(j, acc):
                a1, a2 = acc
                for u in range(4):
                    kv = _key(candv[pl.ds((j * 4 + u) * 16, 16)])
                    a1 = a1 + (kv >= t1).astype(jnp.int32)
                    a2 = a2 + (kv >= t2).astype(jnp.int32)
                return a1, a2
            p1, p2 = lax.fori_loop(
                0, nv4, cb,
                (jnp.zeros((16,), jnp.int32), jnp.zeros((16,), jnp.int32)))
            return (lax.reduce_max(plsc.cumsum(p1), axes=(0,)),
                    lax.reduce_max(plsc.cumsum(p2), axes=(0,)))

        def _cavg(a, b):
            return (a & b) + ((a ^ b) >> 1)

        lo0 = _scalar(_key(tl_vec)) - 1
        hi0 = _scalar(_key(jnp.broadcast_to(gmax, (16,)))) + 1

        def bs_cond(lohi):
            return lohi[0] < lohi[1]

        def bs_body(lohi):
            lo, hi = lohi
            mid = _cavg(lo + 1, hi)
            m1 = _cavg(lo + 1, mid - 1)
            m1 = jnp.maximum(m1, lo + 1)
            c1, c2 = count_ge2(m1, mid)
            ge2 = c2 >= _K
            ge1 = c1 >= _K
            lo = jnp.where(ge2, mid, jnp.where(ge1, m1, lo))
            hi = jnp.where(ge2, hi, jnp.where(ge1, mid - 1, m1 - 1))
            return lo, hi
        t, _unused = lax.while_loop(bs_cond, bs_body, (lo0, hi0))

        def count_ge(tq):
            def cb(j, acc):
                for u in range(4):
                    kv = _key(candv[pl.ds((j * 4 + u) * 16, 16)])
                    acc = acc + (kv >= tq).astype(jnp.int32)
                return acc
            part = lax.fori_loop(0, nv4, cb, jnp.zeros((16,), jnp.int32))
            return lax.reduce_max(plsc.cumsum(part), axes=(0,))

        total = count_ge(t)
        fast = (total == _K) & (t != 0) & (t != -1)

        @pl.when(fast)
        def _():
            tf = _unkey_splat(t)

            def emit_fast(i, carry):
                for u in range(16):
                    sl = pl.ds((i * 16 + u) * 16, 16)
                    x = rowbuf[sl]
                    rowbuf[sl] = jnp.where(x >= tf, x, neg)
                return carry
            lax.fori_loop(0, _NV // 16, emit_fast, 0)

        @pl.when(jnp.logical_not(fast))
        def _():
            m_eq = _K - count_ge(t + 1)

            def emit(i, eq_seen):
                for u in range(4):
                    sl = pl.ds((i * 4 + u) * 16, 16)
                    x = rowbuf[sl]
                    kv = _key(x)
                    eq = kv == t
                    inc = plsc.cumsum(eq.astype(jnp.int32))
                    keep = (kv > t) | (eq & ((eq_seen + inc) <= m_eq))
                    rowbuf[sl] = jnp.where(keep, x, neg)
                    eq_seen = eq_seen + plsc.all_reduce_population_count(eq)[0]
                return eq_seen
            lax.fori_loop(0, _NV // 4, emit, jnp.int32(0))

        out_flight[r] = pltpu.async_copy(
            rowbuf, out_hbm.at[base_row + r], out_sems[r % 2])

    out_flight.pop(_ROWS_PER_TEC - 2).wait()
    out_flight.pop(_ROWS_PER_TEC - 1).wait()


def kernel(scores, k):
    mesh = plsc.VectorSubcoreMesh(core_axis_name="c", subcore_axis_name="s",
                                  num_cores=2, num_subcores=16)
    out = pl.kernel(
        _sc_body,
        out_type=jax.ShapeDtypeStruct((_R, _C), jnp.float32),
        mesh=mesh,
        compiler_params=pltpu.CompilerParams(needs_layout_passes=False),
        scratch_types=[
            pltpu.VMEM((_C,), jnp.float32),
            pltpu.VMEM((_C,), jnp.float32),
            pltpu.VMEM((_C + 16,), jnp.float32),
            pltpu.SemaphoreType.DMA,
            pltpu.SemaphoreType.DMA,
            pltpu.SemaphoreType.DMA,
            pltpu.SemaphoreType.DMA,
        ],
    )(scores)
    return out + (k * 0)
